# baseline (speedup 1.0000x reference)
import functools

import jax
import jax.numpy as jnp
from jax import lax
from jax.experimental import pallas as pl
from jax.experimental.pallas import tpu as pltpu
from jax.experimental.pallas import tpu_sc as plsc

D = 64
DE = 32
DX = 3
NH = 4
N = 512
EDGES = 32768

EB = 1024
NB = EDGES // EB
IB = 32
F32 = jnp.float32


def _dot(a, b):
    return lax.dot_general(a, b, (((1,), (0,)), ((), ())),
                           preferred_element_type=F32)


def _dot_t(a, b):
    return lax.dot_general(a, b, (((0,), (0,)), ((), ())),
                           preferred_element_type=F32)


def _silu(x):
    return x * jax.nn.sigmoid(x)


def _split_bf16(x):
    hi = x.astype(jnp.bfloat16)
    lo = (x - hi.astype(F32)).astype(jnp.bfloat16)
    return hi, lo


def _gather_z(z_pairs, idx3):
    info = plsc.get_sparse_core_info()
    nc, ns = info.num_cores, info.num_subcores
    nw = nc * ns
    per_w = EDGES // nw
    ch = per_w // 128

    mesh = plsc.VectorSubcoreMesh(core_axis_name="c", subcore_axis_name="s")

    @functools.partial(
        pl.kernel, mesh=mesh,
        out_type=jax.ShapeDtypeStruct((EDGES, 2 * D), F32),
        scratch_types=[
            pltpu.VMEM((ch, 128), jnp.int32),
            pltpu.VMEM((128, 2 * D), F32),
            pltpu.SemaphoreType.DMA,
        ],
    )
    def k(z_hbm, idx_hbm, out_hbm, idx_v, rows_v, sem):
        wid = lax.axis_index("s") * nc + lax.axis_index("c")
        pltpu.sync_copy(idx_hbm.at[wid], idx_v)
        base = wid * per_w
        for c in range(ch):
            pltpu.async_copy(z_hbm.at[idx_v.at[c]], rows_v, sem).wait()
            pltpu.sync_copy(rows_v, out_hbm.at[pl.ds(base + c * 128, 128)])

    return k(z_pairs, idx3)


def _node_body(h_ref, qw1, qb1, qw2, qb2, kw1d, kb1, lw1, lb1, lw2, lb2,
               rw1, rb1, rw2, rb2, ew1s, ew1d,
               qn_ref, akv_ref, lz_ref, rz_ref, hsp_ref, hdp_ref):
    h = h_ref[...]
    qn_ref[...] = _dot(_silu(_dot(h, qw1[...]) + qb1[...]), qw2[...]) + qb2[...]
    akv_ref[...] = _dot(h, kw1d[...]) + kb1[...]
    lz_ref[...] = _dot(_silu(_dot(h, lw1[...]) + lb1[...]), lw2[...]) + lb2[...]
    rz_ref[...] = _dot(_silu(_dot(h, rw1[...]) + rb1[...]), rw2[...]) + rb2[...]
    hsp_ref[...] = _dot(h, ew1s[...])
    hdp_ref[...] = _dot(h, ew1d[...])


def _edge1_body(src_ref, dst_ref, par_ref, tsrc_hi_ref, tsrc_lo_ref,
                tdst_hi_ref, tdst_lo_ref, zg_ref, e_ref,
                w1r, kb1, kw2, kb2, pw1, pb1, pw2, pb2,
                gw1, gb1, gw2, gb2, xw1, xb1, xw2, xb2,
                v_ref, aux_ref, den_ref):
    src = src_ref[0, 0, :]
    dst = dst_ref[0, 0, :]
    iota_n = lax.broadcasted_iota(jnp.int32, (EB, N), 1)
    m_s = src[:, None] == iota_n
    m_d = dst[:, None] == iota_n
    s_sb = m_s.astype(jnp.bfloat16)
    s_db = m_d.astype(jnp.bfloat16)

    gs = _dot(s_sb, tsrc_hi_ref[...]) + _dot(s_sb, tsrc_lo_ref[...])
    gd = _dot(s_db, tdst_hi_ref[...]) + _dot(s_db, tdst_lo_ref[...])
    xs, qe = gs[:, :DX], gs[:, DX:]
    xd, ae = gd[:, :DX], gd[:, DX:]

    rel = xs - xd
    rdist = jnp.sum(rel * rel, axis=1, keepdims=True)

    h_kv = _silu(ae + rdist * w1r[...] + kb1[...])
    kv = _dot(h_kv, kw2[...]) + kb2[...]
    k = kv[:, :NH * D]
    v = kv[:, NH * D:]

    qk = qe * k
    scores = jnp.concatenate(
        [jnp.sum(qk[:, h * D:(h + 1) * D], axis=1, keepdims=True)
         for h in range(NH)], axis=1) * (1.0 / 8.0)

    par = par_ref[0, 0, :]
    zrow = zg_ref[...]
    zsel = jnp.where(par[:, None] == 1, zrow[:, D:], zrow[:, :D])
    hb = _silu(_dot(zsel, pw1[...]) + pb1[...])
    scores = scores + _dot(hb, pw2[...]) + pb2[...]
    ex = jnp.exp(scores)

    hg = _silu(_dot(e_ref[...], gw1[...]) + gb1[...])
    gate = jax.nn.sigmoid(_dot(hg, gw2[...]) + gb2[...])

    hx = _silu(_dot(v, xw1[...]) + xb1[...])
    cw = jnp.clip(_dot(hx, xw2[...]) + xb2[...], -10.0, 10.0)

    xrn = rel / (1.0 + jnp.sqrt(rdist + 1e-8))

    v_ref[...] = v
    aux_ref[...] = jnp.concatenate([gate * ex, xrn, cw], axis=1)

    @pl.when(pl.program_id(0) == 0)
    def _():
        den_ref[...] = jnp.zeros_like(den_ref)
    den_ref[...] += _dot_t(m_s.astype(F32), ex)


def _edge2_body(src_ref, dst_ref, den_ref, hsp_hi_ref, hsp_lo_ref,
                hdp_hi_ref, hdp_lo_ref, v_ref, aux_ref,
                e_ref, ew1a, ew1x, eb1, ew2, eb2,
                eout_ref, acc_ref):
    src = src_ref[0, 0, :]
    dst = dst_ref[0, 0, :]
    iota_n = lax.broadcasted_iota(jnp.int32, (EB, N), 1)
    m_s = src[:, None] == iota_n
    m_d = dst[:, None] == iota_n
    s_sb = m_s.astype(jnp.bfloat16)
    s_db = m_d.astype(jnp.bfloat16)

    rd = 1.0 / (den_ref[...] + 1e-16)
    aux = aux_ref[...]
    gex, xrn, cw = aux[:, :NH], aux[:, NH:NH + DX], aux[:, NH + DX:]
    alpha_ij = gex * _dot(m_s.astype(F32), rd)

    v = v_ref[...]
    u = jnp.concatenate(
        [alpha_ij[:, h:h + 1] * v[:, h * D:(h + 1) * D] for h in range(NH)],
        axis=1)
    am = jnp.sum(alpha_ij, axis=1, keepdims=True) * (1.0 / NH)
    xu = am * xrn * cw
    p = jnp.concatenate([u, xu, am], axis=1)

    @pl.when(pl.program_id(0) == 0)
    def _():
        acc_ref[...] = jnp.zeros_like(acc_ref)
    p_hi, p_lo = _split_bf16(p)
    acc_ref[...] += _dot_t(s_db, p_hi) + _dot_t(s_db, p_lo)

    hs_e = _dot(s_sb, hsp_hi_ref[...]) + _dot(s_sb, hsp_lo_ref[...])
    hd_e = _dot(s_db, hdp_hi_ref[...]) + _dot(s_db, hdp_lo_ref[...])
    he = _silu(_dot(alpha_ij, ew1a[...]) + _dot(xrn, ew1x[...])
               + hs_e + hd_e + eb1[...])
    eout_ref[...] = e_ref[...] + _dot(he, ew2[...]) + eb2[...]


def _final_body(h_ref, x_ref, attn_ref, xup_ref, hw1, hb1, hw2, hb2,
                hout_ref, xout_ref):
    hh = _silu(_dot(attn_ref[...], hw1[...]) + hb1[...])
    hout_ref[...] = h_ref[...] + _dot(hh, hw2[...]) + hb2[...]
    xout_ref[...] = x_ref[...] + xup_ref[...]


def _z_body(z_ref, lz_ref, rz_ref, alpha_ref, jw1, jb1, jw2, jb2, zout_ref):
    ar = alpha_ref[...] * rz_ref[...]
    lb = lz_ref[...]
    m = (lb[:, None, :] * ar[None, :, :]).reshape(IB * N, D)
    h = _silu(_dot(m, jw1[...]) + jb1[...])
    zout_ref[...] = (z_ref[...].reshape(IB * N, D)
                     + _dot(h, jw2[...]) + jb2[...]).reshape(IB, N, D)


def _row(x):
    return x.reshape(1, -1)


def kernel(batch, X, H, E, E_idx, Z, params):
    src = E_idx[0]
    dst = E_idx[1]

    flat_idx = (src * (N // 2) + lax.shift_right_logical(dst, 1)).astype(jnp.int32)
    parity = jnp.bitwise_and(dst, 1).astype(jnp.int32)
    zg = _gather_z(Z.reshape(N * N // 2, 2 * D), flat_idx.reshape(32, -1, 128))

    p_q, p_kv = params["Q"], params["KV"]
    p_pb, p_g = params["pair_bias"], params["gate"]
    p_ph, p_px = params["phi_h"], params["phi_x"]
    p_l, p_r, p_j = params["left_z"], params["right_z"], params["joint_z"]
    p_e = params["phi_e"]

    node_out = pl.pallas_call(
        _node_body,
        out_shape=[
            jax.ShapeDtypeStruct((N, NH * D), F32),
            jax.ShapeDtypeStruct((N, 2 * D), F32),
            jax.ShapeDtypeStruct((N, D), F32),
            jax.ShapeDtypeStruct((N, D), F32),
            jax.ShapeDtypeStruct((N, DE), F32),
            jax.ShapeDtypeStruct((N, DE), F32),
        ],
    )(H, p_q["W1"], _row(p_q["b1"]), p_q["W2"], _row(p_q["b2"]),
      p_kv["W1"][1:], _row(p_kv["b1"]),
      p_l["W1"], _row(p_l["b1"]), p_l["W2"], _row(p_l["b2"]),
      p_r["W1"], _row(p_r["b1"]), p_r["W2"], _row(p_r["b2"]),
      p_e["W1"][NH + DX:NH + DX + D], p_e["W1"][NH + DX + D:])
    qn, akv, lz, rz, hsp, hdp = node_out

    tsrc_hi, tsrc_lo = _split_bf16(jnp.concatenate([X, qn], axis=1))
    tdst_hi, tdst_lo = _split_bf16(jnp.concatenate([X, akv], axis=1))
    hsp_hi, hsp_lo = _split_bf16(hsp)
    hdp_hi, hdp_lo = _split_bf16(hdp)
    src3 = src.reshape(NB, 1, EB)
    dst3 = dst.reshape(NB, 1, EB)

    eblk = lambda w: pl.BlockSpec((EB, w), lambda i: (i, 0))
    iblk = pl.BlockSpec((1, 1, EB), lambda i: (i, 0, 0))
    const = lambda s: pl.BlockSpec(s, lambda i: (0,) * len(s))

    v_e, aux, den = pl.pallas_call(
        _edge1_body,
        grid=(NB,),
        in_specs=[iblk, iblk, iblk,
                  const((N, DX + NH * D)), const((N, DX + NH * D)),
                  const((N, DX + 2 * D)), const((N, DX + 2 * D)),
                  eblk(2 * D), eblk(DE),
                  const((1, 2 * D)), const((1, 2 * D)), const((2 * D, 8 * D)),
                  const((1, 8 * D)),
                  const((D, D)), const((1, D)), const((D, NH)), const((1, NH)),
                  const((DE, DE)), const((1, DE)), const((DE, NH)),
                  const((1, NH)),
                  const((NH * D, D)), const((1, D)), const((D, 1)),
                  const((1, 1))],
        out_specs=[eblk(NH * D), eblk(8), const((N, NH))],
        out_shape=[
            jax.ShapeDtypeStruct((EDGES, NH * D), F32),
            jax.ShapeDtypeStruct((EDGES, 8), F32),
            jax.ShapeDtypeStruct((N, NH), F32),
        ],
    )(src3, dst3, parity.reshape(NB, 1, EB),
      tsrc_hi, tsrc_lo, tdst_hi, tdst_lo, zg, E,
      _row(p_kv["W1"][0]), _row(p_kv["b1"]), p_kv["W2"], _row(p_kv["b2"]),
      p_pb["W1"], _row(p_pb["b1"]), p_pb["W2"], _row(p_pb["b2"]),
      p_g["W1"], _row(p_g["b1"]), p_g["W2"], _row(p_g["b2"]),
      p_px["W1"], _row(p_px["b1"]), p_px["W2"], _row(p_px["b2"]))

    e_out, acc = pl.pallas_call(
        _edge2_body,
        grid=(NB,),
        in_specs=[iblk, iblk, const((N, NH)),
                  const((N, DE)), const((N, DE)),
                  const((N, DE)), const((N, DE)),
                  eblk(NH * D), eblk(8), eblk(DE),
                  const((NH, DE)), const((DX, DE)), const((1, DE)),
                  const((DE, DE)), const((1, DE))],
        out_specs=[eblk(DE), const((N, NH * D + DX + 1))],
        out_shape=[
            jax.ShapeDtypeStruct((EDGES, DE), F32),
            jax.ShapeDtypeStruct((N, NH * D + DX + 1), F32),
        ],
    )(src3, dst3, den, hsp_hi, hsp_lo, hdp_hi, hdp_lo, v_e, aux, E,
      p_e["W1"][:NH], p_e["W1"][NH:NH + DX], _row(p_e["b1"]),
      p_e["W2"], _row(p_e["b2"]))

    attn_out = acc[:, :NH * D]
    x_up = acc[:, NH * D:NH * D + DX]
    alpha = acc[:, NH * D + DX:]

    h_out, x_out = pl.pallas_call(
        _final_body,
        out_shape=[jax.ShapeDtypeStruct((N, D), F32),
                   jax.ShapeDtypeStruct((N, DX), F32)],
    )(H, X, attn_out, x_up,
      p_ph["W1"], _row(p_ph["b1"]), p_ph["W2"], _row(p_ph["b2"]))

    z_out = pl.pallas_call(
        _z_body,
        grid=(N // IB,),
        in_specs=[pl.BlockSpec((IB, N, D), lambda i: (i, 0, 0)),
                  pl.BlockSpec((IB, D), lambda i: (i, 0)),
                  const((N, D)), const((N, 1)),
                  const((D, D)), const((1, D)), const((D, D)), const((1, D))],
        out_specs=pl.BlockSpec((IB, N, D), lambda i: (i, 0, 0)),
        out_shape=jax.ShapeDtypeStruct((N, N, D), F32),
    )(Z, lz, rz, alpha,
      p_j["W1"], _row(p_j["b1"]), p_j["W2"], _row(p_j["b2"]))

    return (h_out, x_out, z_out, e_out)


# speedup vs baseline: 1.3880x; 1.3880x over previous
import functools

import jax
import jax.numpy as jnp
from jax import lax
from jax.experimental import pallas as pl
from jax.experimental.pallas import tpu as pltpu
from jax.experimental.pallas import tpu_sc as plsc

D = 64
DE = 32
DX = 3
NH = 4
N = 512
EDGES = 32768

EB = 1024
NB = EDGES // EB
IB = 32
F32 = jnp.float32


def _dot(a, b):
    return lax.dot_general(a, b, (((1,), (0,)), ((), ())),
                           preferred_element_type=F32)


def _dot_t(a, b):
    return lax.dot_general(a, b, (((0,), (0,)), ((), ())),
                           preferred_element_type=F32)


def _silu(x):
    return x * jax.nn.sigmoid(x)


def _split_bf16(x):
    hi = x.astype(jnp.bfloat16)
    lo = (x - hi.astype(F32)).astype(jnp.bfloat16)
    return hi, lo


def _gather_z(z_pairs, idx3):
    info = plsc.get_sparse_core_info()
    nc, ns = info.num_cores, info.num_subcores
    nw = nc * ns
    per_w = EDGES // nw
    ch = per_w // 128

    mesh = plsc.VectorSubcoreMesh(core_axis_name="c", subcore_axis_name="s")

    @functools.partial(
        pl.kernel, mesh=mesh,
        out_type=jax.ShapeDtypeStruct((EDGES, 2 * D), F32),
        scratch_types=[
            pltpu.VMEM((ch, 128), jnp.int32),
            pltpu.VMEM((128, 2 * D), F32),
            pltpu.SemaphoreType.DMA,
        ],
    )
    def k(z_hbm, idx_hbm, out_hbm, idx_v, rows_v, sem):
        wid = lax.axis_index("s") * nc + lax.axis_index("c")
        pltpu.sync_copy(idx_hbm.at[wid], idx_v)
        base = wid * per_w
        for c in range(ch):
            pltpu.async_copy(z_hbm.at[idx_v.at[c]], rows_v, sem).wait()
            pltpu.sync_copy(rows_v, out_hbm.at[pl.ds(base + c * 128, 128)])

    return k(z_pairs, idx3)


def _node_body(h_ref, qw1, qb1, qw2, qb2, kw1d, kb1, lw1, lb1, lw2, lb2,
               rw1, rb1, rw2, rb2, ew1s, ew1d,
               qn_ref, akv_ref, lz_ref, rz_ref, hsp_ref, hdp_ref):
    h = h_ref[...]
    qn_ref[...] = _dot(_silu(_dot(h, qw1[...]) + qb1[...]), qw2[...]) + qb2[...]
    akv_ref[...] = _dot(h, kw1d[...]) + kb1[...]
    lz_ref[...] = _dot(_silu(_dot(h, lw1[...]) + lb1[...]), lw2[...]) + lb2[...]
    rz_ref[...] = _dot(_silu(_dot(h, rw1[...]) + rb1[...]), rw2[...]) + rb2[...]
    hsp_ref[...] = _dot(h, ew1s[...])
    hdp_ref[...] = _dot(h, ew1d[...])


def _edge1_body(src_ref, dst_ref, par_ref, tsrc_ref, tdst_ref, zg_ref, e_ref,
                w1r, kb1, kw2, kb2, pw1, pb1, pw2, pb2,
                gw1, gb1, gw2, gb2, xw1, xb1, xw2, xb2,
                v_ref, aux_ref, den_ref):
    src = src_ref[0, 0, :]
    dst = dst_ref[0, 0, :]
    iota_n = lax.broadcasted_iota(jnp.int32, (EB, N), 1)
    s_s = (src[:, None] == iota_n).astype(F32)
    s_d = (dst[:, None] == iota_n).astype(F32)

    gs = _dot(s_s, tsrc_ref[...])
    gd = _dot(s_d, tdst_ref[...])
    xs, qe = gs[:, :DX], gs[:, DX:]
    xd, ae = gd[:, :DX], gd[:, DX:]

    rel = xs - xd
    rdist = jnp.sum(rel * rel, axis=1, keepdims=True)

    h_kv = _silu(ae + rdist * w1r[...] + kb1[...])
    kv = _dot(h_kv, kw2[...]) + kb2[...]
    k = kv[:, :NH * D]
    v = kv[:, NH * D:]

    qk = qe * k
    scores = jnp.concatenate(
        [jnp.sum(qk[:, h * D:(h + 1) * D], axis=1, keepdims=True)
         for h in range(NH)], axis=1) * (1.0 / 8.0)

    par = par_ref[0, 0, :]
    zrow = zg_ref[...]
    zsel = jnp.where(par[:, None] == 1, zrow[:, D:], zrow[:, :D])
    hb = _silu(_dot(zsel, pw1[...]) + pb1[...])
    scores = scores + _dot(hb, pw2[...]) + pb2[...]
    ex = jnp.exp(scores)

    hg = _silu(_dot(e_ref[...], gw1[...]) + gb1[...])
    gate = jax.nn.sigmoid(_dot(hg, gw2[...]) + gb2[...])

    hx = _silu(_dot(v, xw1[...]) + xb1[...])
    cw = jnp.clip(_dot(hx, xw2[...]) + xb2[...], -10.0, 10.0)

    xrn = rel / (1.0 + jnp.sqrt(rdist + 1e-8))

    v_ref[...] = v
    aux_ref[...] = jnp.concatenate([gate * ex, xrn, cw], axis=1)

    @pl.when(pl.program_id(0) == 0)
    def _():
        den_ref[...] = jnp.zeros_like(den_ref)
    den_ref[...] += _dot_t(s_s, ex)


def _edge2_body(src_ref, dst_ref, den_ref, hsp_ref, hdp_ref, v_ref, aux_ref,
                e_ref, ew1a, ew1x, eb1, ew2, eb2,
                eout_ref, acc_ref):
    src = src_ref[0, 0, :]
    dst = dst_ref[0, 0, :]
    iota_n = lax.broadcasted_iota(jnp.int32, (EB, N), 1)
    s_s = (src[:, None] == iota_n).astype(F32)
    s_d = (dst[:, None] == iota_n).astype(F32)

    rd = 1.0 / (den_ref[...] + 1e-16)
    aux = aux_ref[...]
    gex, xrn, cw = aux[:, :NH], aux[:, NH:NH + DX], aux[:, NH + DX:]
    alpha_ij = gex * _dot(s_s, rd)

    v = v_ref[...]
    u = jnp.concatenate(
        [alpha_ij[:, h:h + 1] * v[:, h * D:(h + 1) * D] for h in range(NH)],
        axis=1)
    am = jnp.sum(alpha_ij, axis=1, keepdims=True) * (1.0 / NH)
    xu = am * xrn * cw
    p = jnp.concatenate([u, xu, am], axis=1)

    @pl.when(pl.program_id(0) == 0)
    def _():
        acc_ref[...] = jnp.zeros_like(acc_ref)
    acc_ref[...] += _dot_t(s_d, p)

    he = _silu(_dot(alpha_ij, ew1a[...]) + _dot(xrn, ew1x[...])
               + _dot(s_s, hsp_ref[...]) + _dot(s_d, hdp_ref[...]) + eb1[...])
    eout_ref[...] = e_ref[...] + _dot(he, ew2[...]) + eb2[...]


def _final_body(h_ref, x_ref, attn_ref, xup_ref, hw1, hb1, hw2, hb2,
                hout_ref, xout_ref):
    hh = _silu(_dot(attn_ref[...], hw1[...]) + hb1[...])
    hout_ref[...] = h_ref[...] + _dot(hh, hw2[...]) + hb2[...]
    xout_ref[...] = x_ref[...] + xup_ref[...]


def _z_body(zt_ref, lz_ref, rzt_ref, alphat_ref, jw1t, jb1, jw2t, jb2,
            zout_ref):
    art = alphat_ref[...] * rzt_ref[...]
    lb = lz_ref[...]
    mt = lb[:, :, None] * art[None, :, :]
    w1t = jnp.broadcast_to(jw1t[...], (IB, D, D))
    h = _silu(lax.dot_general(w1t, mt, (((2,), (1,)), ((0,), (0,))),
                              preferred_element_type=F32) + jb1[...])
    w2t = jnp.broadcast_to(jw2t[...], (IB, D, D))
    upd = lax.dot_general(w2t, h, (((2,), (1,)), ((0,), (0,))),
                          preferred_element_type=F32) + jb2[...]
    zout_ref[...] = zt_ref[...] + upd


def _row(x):
    return x.reshape(1, -1)


def kernel(batch, X, H, E, E_idx, Z, params):
    src = E_idx[0]
    dst = E_idx[1]

    flat_idx = (src * (N // 2) + lax.shift_right_logical(dst, 1)).astype(jnp.int32)
    parity = jnp.bitwise_and(dst, 1).astype(jnp.int32)
    zg = _gather_z(Z.reshape(N * N // 2, 2 * D), flat_idx.reshape(32, -1, 128))

    p_q, p_kv = params["Q"], params["KV"]
    p_pb, p_g = params["pair_bias"], params["gate"]
    p_ph, p_px = params["phi_h"], params["phi_x"]
    p_l, p_r, p_j = params["left_z"], params["right_z"], params["joint_z"]
    p_e = params["phi_e"]

    node_out = pl.pallas_call(
        _node_body,
        out_shape=[
            jax.ShapeDtypeStruct((N, NH * D), F32),
            jax.ShapeDtypeStruct((N, 2 * D), F32),
            jax.ShapeDtypeStruct((N, D), F32),
            jax.ShapeDtypeStruct((N, D), F32),
            jax.ShapeDtypeStruct((N, DE), F32),
            jax.ShapeDtypeStruct((N, DE), F32),
        ],
    )(H, p_q["W1"], _row(p_q["b1"]), p_q["W2"], _row(p_q["b2"]),
      p_kv["W1"][1:], _row(p_kv["b1"]),
      p_l["W1"], _row(p_l["b1"]), p_l["W2"], _row(p_l["b2"]),
      p_r["W1"], _row(p_r["b1"]), p_r["W2"], _row(p_r["b2"]),
      p_e["W1"][NH + DX:NH + DX + D], p_e["W1"][NH + DX + D:])
    qn, akv, lz, rz, hsp, hdp = node_out

    tsrc = jnp.concatenate([X, qn], axis=1)
    tdst = jnp.concatenate([X, akv], axis=1)
    src3 = src.reshape(NB, 1, EB)
    dst3 = dst.reshape(NB, 1, EB)

    eblk = lambda w: pl.BlockSpec((EB, w), lambda i: (i, 0))
    iblk = pl.BlockSpec((1, 1, EB), lambda i: (i, 0, 0))
    const = lambda s: pl.BlockSpec(s, lambda i: (0,) * len(s))

    v_e, aux, den = pl.pallas_call(
        _edge1_body,
        grid=(NB,),
        in_specs=[iblk, iblk, iblk,
                  const((N, DX + NH * D)), const((N, DX + 2 * D)),
                  eblk(2 * D), eblk(DE),
                  const((1, 2 * D)), const((1, 2 * D)), const((2 * D, 8 * D)),
                  const((1, 8 * D)),
                  const((D, D)), const((1, D)), const((D, NH)), const((1, NH)),
                  const((DE, DE)), const((1, DE)), const((DE, NH)),
                  const((1, NH)),
                  const((NH * D, D)), const((1, D)), const((D, 1)),
                  const((1, 1))],
        out_specs=[eblk(NH * D), eblk(8), const((N, NH))],
        out_shape=[
            jax.ShapeDtypeStruct((EDGES, NH * D), F32),
            jax.ShapeDtypeStruct((EDGES, 8), F32),
            jax.ShapeDtypeStruct((N, NH), F32),
        ],
    )(src3, dst3, parity.reshape(NB, 1, EB), tsrc, tdst, zg, E,
      _row(p_kv["W1"][0]), _row(p_kv["b1"]), p_kv["W2"], _row(p_kv["b2"]),
      p_pb["W1"], _row(p_pb["b1"]), p_pb["W2"], _row(p_pb["b2"]),
      p_g["W1"], _row(p_g["b1"]), p_g["W2"], _row(p_g["b2"]),
      p_px["W1"], _row(p_px["b1"]), p_px["W2"], _row(p_px["b2"]))

    e_out, acc = pl.pallas_call(
        _edge2_body,
        grid=(NB,),
        in_specs=[iblk, iblk, const((N, NH)), const((N, DE)), const((N, DE)),
                  eblk(NH * D), eblk(8), eblk(DE),
                  const((NH, DE)), const((DX, DE)), const((1, DE)),
                  const((DE, DE)), const((1, DE))],
        out_specs=[eblk(DE), const((N, NH * D + DX + 1))],
        out_shape=[
            jax.ShapeDtypeStruct((EDGES, DE), F32),
            jax.ShapeDtypeStruct((N, NH * D + DX + 1), F32),
        ],
    )(src3, dst3, den, hsp, hdp, v_e, aux, E,
      p_e["W1"][:NH], p_e["W1"][NH:NH + DX], _row(p_e["b1"]),
      p_e["W2"], _row(p_e["b2"]))

    attn_out = acc[:, :NH * D]
    x_up = acc[:, NH * D:NH * D + DX]
    alpha = acc[:, NH * D + DX:]

    h_out, x_out = pl.pallas_call(
        _final_body,
        out_shape=[jax.ShapeDtypeStruct((N, D), F32),
                   jax.ShapeDtypeStruct((N, DX), F32)],
    )(H, X, attn_out, x_up,
      p_ph["W1"], _row(p_ph["b1"]), p_ph["W2"], _row(p_ph["b2"]))

    zt = jnp.transpose(Z, (0, 2, 1))
    zt_out = pl.pallas_call(
        _z_body,
        grid=(N // IB,),
        in_specs=[pl.BlockSpec((IB, D, N), lambda i: (i, 0, 0)),
                  pl.BlockSpec((IB, D), lambda i: (i, 0)),
                  const((D, N)), const((1, N)),
                  const((D, D)), const((D, 1)), const((D, D)), const((D, 1))],
        out_specs=pl.BlockSpec((IB, D, N), lambda i: (i, 0, 0)),
        out_shape=jax.ShapeDtypeStruct((N, D, N), F32),
    )(zt, lz, rz.T, alpha.reshape(1, N),
      p_j["W1"].T, p_j["b1"].reshape(D, 1), p_j["W2"].T,
      p_j["b2"].reshape(D, 1))
    z_out = jnp.transpose(zt_out, (0, 2, 1))

    return (h_out, x_out, z_out, e_out)
